# baseline (speedup 1.0000x reference)
import functools
import math

import jax
import jax.numpy as jnp
from jax import lax
from jax.experimental import pallas as pl
from jax.experimental.pallas import tpu as pltpu
from jax.experimental.pallas import tpu_sc as plsc


def _score_kernel(q_ref, kt_ref, s_ref):
  H, T, D = q_ref.shape
  inv = 1.0 / math.sqrt(D)
  for h in range(H):
    s_ref[h] = jnp.dot(q_ref[h], kt_ref[h],
                       preferred_element_type=jnp.float32) * inv
  raw = s_ref[...]
  m = jnp.max(raw, axis=0, keepdims=True)
  p = jnp.exp(raw - m)
  denom = jnp.sum(p, axis=0, keepdims=True)
  s_ref[...] = p / denom


def _scores(q2, kt2, T):
  H, L, D = q2.shape
  grid = L // T
  return pl.pallas_call(
      _score_kernel,
      grid=(grid,),
      in_specs=[
          pl.BlockSpec((H, T, D), lambda i: (0, i, 0)),
          pl.BlockSpec((H, D, L), lambda i: (0, 0, 0)),
      ],
      out_specs=pl.BlockSpec((H, T, L), lambda i: (0, i, 0)),
      out_shape=jax.ShapeDtypeStruct((H, L, L), jnp.float32),
  )(q2, kt2)


def _topk_kernel(s_ref, idx_ref, *, num, rows_per_head, blocks_per_head):
  R, L = s_ref.shape
  CAND = 128
  NITER = 10
  v = s_ref[...]

  rmax = jnp.max(v, axis=1)
  mean = jnp.sum(v, axis=1) * (1.0 / L)
  ex2 = jnp.sum(v * v, axis=1) * (1.0 / L)
  sig = jnp.sqrt(jnp.maximum(ex2 - mean * mean, 0.0))

  lo = jnp.zeros((R,), jnp.float32)
  cnt_lo = jnp.full((R,), float(L), jnp.float32)
  hi = rmax
  cnt_hi = jnp.ones((R,), jnp.float32)
  t_found = jnp.zeros((R,), jnp.float32)
  done = jnp.zeros((R,), jnp.bool_)
  guess = jnp.minimum(mean + 1.5 * sig, rmax * 0.5)
  target = 0.5 * (num + CAND)
  for it in range(NITER):
    if it == 0:
      mid = guess
    elif it % 2 == 1:
      frac = (cnt_lo - target) / jnp.maximum(cnt_lo - cnt_hi, 1.0)
      mid = lo + (hi - lo) * jnp.clip(frac, 0.03, 0.97)
    else:
      mid = 0.5 * (lo + hi)
    cnt = jnp.sum((v >= mid[:, None]).astype(jnp.float32), axis=1)
    ok = (cnt >= num) & (cnt <= CAND)
    t_found = jnp.where(ok & ~done, mid, t_found)
    too_low = cnt > CAND
    upd = ~(done | ok)
    lo = jnp.where(upd & too_low, mid, lo)
    cnt_lo = jnp.where(upd & too_low, cnt, cnt_lo)
    hi = jnp.where(upd & ~too_low, mid, hi)
    cnt_hi = jnp.where(upd & ~too_low, cnt, cnt_hi)
    done = done | ok
  thr = jnp.where(done, t_found, lo)

  maskf = (v >= thr[:, None]).astype(jnp.float32)
  C = L // 128
  cs = maskf.reshape(R, C, 128)
  lane = lax.broadcasted_iota(jnp.int32, (R, C, 128), 2)
  for s in (1, 2, 4, 8, 16, 32, 64):
    sh = jnp.roll(cs, s, axis=2)
    cs = cs + jnp.where(lane >= s, sh, 0.0)
  chunk_tot = cs[:, :, 127]
  ci = lax.broadcasted_iota(jnp.int32, (R, C), 1)
  incl = chunk_tot
  for s in (1, 2, 4, 8):
    incl = incl + jnp.where(ci >= s, jnp.roll(incl, s, axis=1), 0.0)
  excl = incl - chunk_tot
  csl = cs.reshape(R, L)
  cnt_tot = incl[:, C - 1]

  def _gather_wide(tab, idx):
    acc = jnp.zeros(idx.shape, tab.dtype)
    cid = lax.shift_right_logical(idx, 7)
    lid = jnp.bitwise_and(idx, 127)
    for c in range(tab.shape[1] // 128):
      g = jnp.take_along_axis(tab[:, c * 128:(c + 1) * 128], lid, axis=1,
                              mode="promise_in_bounds")
      acc = jnp.where(cid == c, g, acc)
    return acc

  big = jnp.float32(1e9)
  padw = 128 - C
  pad = jnp.full((R, padw), big, jnp.float32)
  incl_pad = jnp.concatenate([incl, pad], axis=1)
  excl_pad = jnp.concatenate([excl, pad], axis=1)

  slot = lax.broadcasted_iota(jnp.int32, (R, CAND), 1)
  rankf = (slot + 1).astype(jnp.float32)
  c_sel = jnp.zeros((R, CAND), jnp.int32)
  for s in (8, 4, 2, 1):
    cand_c = c_sel + s
    probe = jnp.take_along_axis(incl_pad, cand_c - 1, axis=1,
                                mode="promise_in_bounds")
    c_sel = jnp.where(probe < rankf, cand_c, c_sel)
  excl_c = jnp.take_along_axis(excl_pad, c_sel, axis=1,
                               mode="promise_in_bounds")
  rl = rankf - excl_c
  u = jnp.zeros((R, CAND), jnp.int32)
  cbase = c_sel * 128
  for s in (64, 32, 16, 8, 4, 2, 1):
    cand_u = u + s
    probe = _gather_wide(csl, cbase + cand_u - 1)
    u = jnp.where(probe < rl, cand_u, u)
  j = cbase + u

  vals_c = _gather_wide(v, j)
  valid = slot < cnt_tot[:, None].astype(jnp.int32)
  vals_c = jnp.where(valid, vals_c, -1.0)
  idx_c = jnp.where(valid, j, -1)

  lane1 = lax.broadcasted_iota(jnp.int32, (R, CAND), 1)
  k = 2
  while k <= CAND:
    s = k // 2
    while s >= 1:
      bit = (lane1 & s) != 0
      pv = jnp.where(bit, jnp.roll(vals_c, s, axis=1),
                     jnp.roll(vals_c, -s, axis=1))
      pi = jnp.where(bit, jnp.roll(idx_c, s, axis=1),
                     jnp.roll(idx_c, -s, axis=1))
      t_gt = (pv > vals_c) | ((pv == vals_c) & (pi > idx_c))
      eq = (pv == vals_c) & (pi == idx_c)
      t_lt = ~(t_gt | eq)
      asc = (lane1 & k) == 0
      i_am_low = ~bit
      want_small = i_am_low == asc
      take_p = (want_small & t_lt) | (~want_small & t_gt)
      vals_c = jnp.where(take_p, pv, vals_c)
      idx_c = jnp.where(take_p, pi, idx_c)
      s //= 2
    k *= 2

  head = pl.program_id(0) // blocks_per_head
  out = jnp.roll(idx_c, num - CAND, axis=1) + head * rows_per_head
  idx_ref[...] = jnp.maximum(out, 0)


def _topk(score2d, num):
  NR, L = score2d.shape
  R = 256
  grid = NR // R
  rows_per_head = L
  blocks_per_head = L // R
  kfn = functools.partial(_topk_kernel, num=num,
                          rows_per_head=rows_per_head,
                          blocks_per_head=blocks_per_head)
  return pl.pallas_call(
      kfn,
      grid=(grid,),
      in_specs=[pl.BlockSpec((R, L), lambda i: (i, 0))],
      out_specs=pl.BlockSpec((R, 128), lambda i: (i, 0)),
      out_shape=jax.ShapeDtypeStruct((NR, 128), jnp.int32),
  )(score2d)


def _gather_rows(table, idx_rows, num, H, L):
  NC, NS = 2, 16
  NW = NC * NS
  NR, _ = idx_rows.shape
  D = table.shape[1]
  rows_per_w = NR // NW
  half = rows_per_w // 2
  pairs = half // 2
  mesh = plsc.VectorSubcoreMesh(core_axis_name="c", subcore_axis_name="s")

  @functools.partial(
      pl.kernel,
      mesh=mesh,
      compiler_params=pltpu.CompilerParams(use_tc_tiling_on_sc=False),
      out_type=jax.ShapeDtypeStruct((H, L, num, D), jnp.float32),
      scratch_types=[
          pltpu.VMEM((half, 128), jnp.int32),
          pltpu.VMEM((208, D), jnp.float32),
          pltpu.VMEM((208, D), jnp.float32),
          pltpu.SemaphoreType.DMA,
          pltpu.SemaphoreType.DMA,
      ],
  )
  def gather_kernel(table_hbm, idx_hbm, out_hbm, idx_v, buf0, buf1, sem0, sem1):
    wid = lax.axis_index("s") * NC + lax.axis_index("c")
    rbase = wid * rows_per_w

    npad = 104

    def fire(r, buf, sem):
      pltpu.make_async_copy(table_hbm.at[idx_v.at[2 * r, pl.ds(0, npad)]],
                            buf.at[pl.ds(0, npad)], sem).start()
      pltpu.make_async_copy(table_hbm.at[idx_v.at[2 * r + 1, pl.ds(0, npad)]],
                            buf.at[pl.ds(npad, npad)], sem).start()

    def drain(r, buf, sem):
      pltpu.make_async_copy(table_hbm.at[idx_v.at[2 * r, pl.ds(0, npad)]],
                            buf.at[pl.ds(0, npad)], sem).wait()
      pltpu.make_async_copy(table_hbm.at[idx_v.at[2 * r + 1, pl.ds(0, npad)]],
                            buf.at[pl.ds(npad, npad)], sem).wait()

    def run_half(h):
      pltpu.sync_copy(idx_hbm.at[pl.ds(rbase + h * half, half)], idx_v)
      qbase = rbase + h * half

      def put(r, buf):
        q0 = qbase + 2 * r
        pltpu.sync_copy(buf.at[pl.ds(0, num)],
                        out_hbm.at[q0 // L, q0 % L])
        pltpu.sync_copy(buf.at[pl.ds(npad, num)],
                        out_hbm.at[(q0 + 1) // L, (q0 + 1) % L])

      fire(0, buf0, sem0)

      def body(i, carry):
        r0 = i * 2
        fire(r0 + 1, buf1, sem1)
        drain(r0, buf0, sem0)
        put(r0, buf0)

        @pl.when(r0 + 2 < pairs)
        def _():
          fire(r0 + 2, buf0, sem0)

        drain(r0 + 1, buf1, sem1)
        put(r0 + 1, buf1)
        return carry

      lax.fori_loop(0, pairs // 2, body, 0)

    run_half(0)
    run_half(1)

  return gather_kernel(table, idx_rows)


def _assemble_kernel(rows_ref, out_ref):
  _, Tq, num, D = out_ref.shape
  out_ref[0] = rows_ref[:, :D].reshape(Tq, num, D)


def _assemble(rows, H, L, num, D):
  Tq = 128
  nb = L // Tq
  return pl.pallas_call(
      _assemble_kernel,
      grid=(H, nb),
      in_specs=[pl.BlockSpec((Tq * num, 128), lambda h, i: (h * nb + i, 0))],
      out_specs=pl.BlockSpec((1, Tq, num, D), lambda h, i: (h, i, 0, 0)),
      out_shape=jax.ShapeDtypeStruct((H, L, num, D), jnp.float32),
  )(rows)


def kernel(q, k, v, num, e):
  del num, e
  num = 100
  B, H, L, D = k.shape
  kt = jnp.reshape(k, (B, H, D, L))
  q2 = q[0]
  kt2 = kt[0]

  score = _scores(q2, kt2, T=128)

  idx_full = _topk(score.reshape(H * L, L), num)

  table = v[0].reshape(H * L, D)
  gathered = _gather_rows(table, idx_full, num, H, L)
  return (gathered.reshape(B, H, L, num, D), score.reshape(B, H, L, L))


# speedup vs baseline: 1.0694x; 1.0694x over previous
import functools
import math

import jax
import jax.numpy as jnp
from jax import lax
from jax.experimental import pallas as pl
from jax.experimental.pallas import tpu as pltpu
from jax.experimental.pallas import tpu_sc as plsc


def _score_kernel(q_ref, kt_ref, s_ref):
  H, T, D = q_ref.shape
  inv = 1.0 / math.sqrt(D)
  for h in range(H):
    s_ref[h] = jnp.dot(q_ref[h], kt_ref[h],
                       preferred_element_type=jnp.float32) * inv
  raw = s_ref[...]
  m = jnp.max(raw, axis=0, keepdims=True)
  p = jnp.exp(raw - m)
  denom = jnp.sum(p, axis=0, keepdims=True)
  s_ref[...] = p / denom


def _scores(q2, kt2, T):
  H, L, D = q2.shape
  grid = L // T
  return pl.pallas_call(
      _score_kernel,
      grid=(grid,),
      in_specs=[
          pl.BlockSpec((H, T, D), lambda i: (0, i, 0)),
          pl.BlockSpec((H, D, L), lambda i: (0, 0, 0)),
      ],
      out_specs=pl.BlockSpec((H, T, L), lambda i: (0, i, 0)),
      out_shape=jax.ShapeDtypeStruct((H, L, L), jnp.float32),
  )(q2, kt2)


def _topk_kernel(s_ref, idx_ref, *, num, rows_per_head, blocks_per_head):
  R, L = s_ref.shape
  CAND = 128
  NITER = 8
  v = s_ref[...]

  rmax = jnp.max(v, axis=1)
  mean = jnp.sum(v, axis=1) * (1.0 / L)
  ex2 = jnp.sum(v * v, axis=1) * (1.0 / L)
  sig = jnp.sqrt(jnp.maximum(ex2 - mean * mean, 0.0))

  lo = jnp.zeros((R,), jnp.float32)
  cnt_lo = jnp.full((R,), float(L), jnp.float32)
  hi = rmax
  cnt_hi = jnp.ones((R,), jnp.float32)
  t_found = jnp.zeros((R,), jnp.float32)
  done = jnp.zeros((R,), jnp.bool_)
  guess = jnp.minimum(mean + 1.5 * sig, rmax * 0.5)
  target = 0.5 * (num + CAND)
  for it in range(NITER):
    if it == 0:
      mid = guess
    elif it % 2 == 1:
      frac = (cnt_lo - target) / jnp.maximum(cnt_lo - cnt_hi, 1.0)
      mid = lo + (hi - lo) * jnp.clip(frac, 0.03, 0.97)
    else:
      mid = 0.5 * (lo + hi)
    cnt = jnp.sum((v >= mid[:, None]).astype(jnp.float32), axis=1)
    ok = (cnt >= num) & (cnt <= CAND)
    t_found = jnp.where(ok & ~done, mid, t_found)
    too_low = cnt > CAND
    upd = ~(done | ok)
    lo = jnp.where(upd & too_low, mid, lo)
    cnt_lo = jnp.where(upd & too_low, cnt, cnt_lo)
    hi = jnp.where(upd & ~too_low, mid, hi)
    cnt_hi = jnp.where(upd & ~too_low, cnt, cnt_hi)
    done = done | ok
  thr = jnp.where(done, t_found, lo)

  maskf = (v >= thr[:, None]).astype(jnp.float32)
  C = L // 128
  tri_r = lax.broadcasted_iota(jnp.int32, (128, 128), 0)
  tri_c = lax.broadcasted_iota(jnp.int32, (128, 128), 1)
  tri = (tri_r <= tri_c).astype(jnp.float32)
  cs = jnp.dot(maskf.reshape(R * C, 128), tri,
               preferred_element_type=jnp.float32).reshape(R, C, 128)
  chunk_tot = cs[:, :, 127]
  ci = lax.broadcasted_iota(jnp.int32, (R, C), 1)
  incl = chunk_tot
  for s in (1, 2, 4, 8):
    incl = incl + jnp.where(ci >= s, jnp.roll(incl, s, axis=1), 0.0)
  excl = incl - chunk_tot
  csl = cs.reshape(R, L)
  cnt_tot = incl[:, C - 1]

  def _gather_wide(tab, idx):
    acc = jnp.zeros(idx.shape, tab.dtype)
    cid = lax.shift_right_logical(idx, 7)
    lid = jnp.bitwise_and(idx, 127)
    for c in range(tab.shape[1] // 128):
      g = jnp.take_along_axis(tab[:, c * 128:(c + 1) * 128], lid, axis=1,
                              mode="promise_in_bounds")
      acc = jnp.where(cid == c, g, acc)
    return acc

  big = jnp.float32(1e9)
  padw = 128 - C
  pad = jnp.full((R, padw), big, jnp.float32)
  incl_pad = jnp.concatenate([incl, pad], axis=1)
  excl_pad = jnp.concatenate([excl, pad], axis=1)

  slot = lax.broadcasted_iota(jnp.int32, (R, CAND), 1)
  rankf = (slot + 1).astype(jnp.float32)
  c_sel = jnp.zeros((R, CAND), jnp.int32)
  for s in (8, 4, 2, 1):
    cand_c = c_sel + s
    probe = jnp.take_along_axis(incl_pad, cand_c - 1, axis=1,
                                mode="promise_in_bounds")
    c_sel = jnp.where(probe < rankf, cand_c, c_sel)
  excl_c = jnp.take_along_axis(excl_pad, c_sel, axis=1,
                               mode="promise_in_bounds")
  rl = rankf - excl_c
  u = jnp.zeros((R, CAND), jnp.int32)
  cbase = c_sel * 128
  for s in (64, 32, 16, 8, 4, 2, 1):
    cand_u = u + s
    probe = _gather_wide(csl, cbase + cand_u - 1)
    u = jnp.where(probe < rl, cand_u, u)
  j = cbase + u

  vals_c = _gather_wide(v, j)
  valid = slot < cnt_tot[:, None].astype(jnp.int32)
  vals_c = jnp.where(valid, vals_c, -1.0)
  idx_c = jnp.where(valid, j, -1)

  lane1 = lax.broadcasted_iota(jnp.int32, (R, CAND), 1)
  k = 2
  while k <= CAND:
    s = k // 2
    while s >= 1:
      bit = (lane1 & s) != 0
      pv = jnp.where(bit, jnp.roll(vals_c, s, axis=1),
                     jnp.roll(vals_c, -s, axis=1))
      pi = jnp.where(bit, jnp.roll(idx_c, s, axis=1),
                     jnp.roll(idx_c, -s, axis=1))
      t_gt = (pv > vals_c) | ((pv == vals_c) & (pi > idx_c))
      eq = (pv == vals_c) & (pi == idx_c)
      t_lt = ~(t_gt | eq)
      asc = (lane1 & k) == 0
      i_am_low = ~bit
      want_small = i_am_low == asc
      take_p = (want_small & t_lt) | (~want_small & t_gt)
      vals_c = jnp.where(take_p, pv, vals_c)
      idx_c = jnp.where(take_p, pi, idx_c)
      s //= 2
    k *= 2

  head = pl.program_id(0) // blocks_per_head
  out = jnp.roll(idx_c, num - CAND, axis=1) + head * rows_per_head
  idx_ref[...] = jnp.maximum(out, 0)


def _topk(score2d, num):
  NR, L = score2d.shape
  R = 256
  grid = NR // R
  rows_per_head = L
  blocks_per_head = L // R
  kfn = functools.partial(_topk_kernel, num=num,
                          rows_per_head=rows_per_head,
                          blocks_per_head=blocks_per_head)
  return pl.pallas_call(
      kfn,
      grid=(grid,),
      in_specs=[pl.BlockSpec((R, L), lambda i: (i, 0))],
      out_specs=pl.BlockSpec((R, 128), lambda i: (i, 0)),
      out_shape=jax.ShapeDtypeStruct((NR, 128), jnp.int32),
  )(score2d)


def _gather_rows(table, idx_rows, num, H, L):
  NC, NS = 2, 16
  NW = NC * NS
  NR, _ = idx_rows.shape
  D = table.shape[1]
  rows_per_w = NR // NW
  half = rows_per_w // 2
  pairs = half // 2
  mesh = plsc.VectorSubcoreMesh(core_axis_name="c", subcore_axis_name="s")

  @functools.partial(
      pl.kernel,
      mesh=mesh,
      compiler_params=pltpu.CompilerParams(use_tc_tiling_on_sc=False),
      out_type=jax.ShapeDtypeStruct((H, L, num, D), jnp.float32),
      scratch_types=[
          pltpu.VMEM((half, 128), jnp.int32),
          pltpu.VMEM((208, D), jnp.float32),
          pltpu.VMEM((208, D), jnp.float32),
          pltpu.SemaphoreType.DMA,
          pltpu.SemaphoreType.DMA,
      ],
  )
  def gather_kernel(table_hbm, idx_hbm, out_hbm, idx_v, buf0, buf1, sem0, sem1):
    wid = lax.axis_index("s") * NC + lax.axis_index("c")
    rbase = wid * rows_per_w

    npad = 104

    def fire(r, buf, sem):
      pltpu.make_async_copy(table_hbm.at[idx_v.at[2 * r, pl.ds(0, npad)]],
                            buf.at[pl.ds(0, npad)], sem).start()
      pltpu.make_async_copy(table_hbm.at[idx_v.at[2 * r + 1, pl.ds(0, npad)]],
                            buf.at[pl.ds(npad, npad)], sem).start()

    def drain(r, buf, sem):
      pltpu.make_async_copy(table_hbm.at[idx_v.at[2 * r, pl.ds(0, npad)]],
                            buf.at[pl.ds(0, npad)], sem).wait()
      pltpu.make_async_copy(table_hbm.at[idx_v.at[2 * r + 1, pl.ds(0, npad)]],
                            buf.at[pl.ds(npad, npad)], sem).wait()

    def run_half(h):
      pltpu.sync_copy(idx_hbm.at[pl.ds(rbase + h * half, half)], idx_v)
      qbase = rbase + h * half

      def put(r, buf):
        q0 = qbase + 2 * r
        pltpu.sync_copy(buf.at[pl.ds(0, num)],
                        out_hbm.at[q0 // L, q0 % L])
        pltpu.sync_copy(buf.at[pl.ds(npad, num)],
                        out_hbm.at[(q0 + 1) // L, (q0 + 1) % L])

      fire(0, buf0, sem0)

      def body(i, carry):
        r0 = i * 2
        fire(r0 + 1, buf1, sem1)
        drain(r0, buf0, sem0)
        put(r0, buf0)

        @pl.when(r0 + 2 < pairs)
        def _():
          fire(r0 + 2, buf0, sem0)

        drain(r0 + 1, buf1, sem1)
        put(r0 + 1, buf1)
        return carry

      lax.fori_loop(0, pairs // 2, body, 0)

    run_half(0)
    run_half(1)

  return gather_kernel(table, idx_rows)


def _assemble_kernel(rows_ref, out_ref):
  _, Tq, num, D = out_ref.shape
  out_ref[0] = rows_ref[:, :D].reshape(Tq, num, D)


def _assemble(rows, H, L, num, D):
  Tq = 128
  nb = L // Tq
  return pl.pallas_call(
      _assemble_kernel,
      grid=(H, nb),
      in_specs=[pl.BlockSpec((Tq * num, 128), lambda h, i: (h * nb + i, 0))],
      out_specs=pl.BlockSpec((1, Tq, num, D), lambda h, i: (h, i, 0, 0)),
      out_shape=jax.ShapeDtypeStruct((H, L, num, D), jnp.float32),
  )(rows)


def kernel(q, k, v, num, e):
  del num, e
  num = 100
  B, H, L, D = k.shape
  kt = jnp.reshape(k, (B, H, D, L))
  q2 = q[0]
  kt2 = kt[0]

  score = _scores(q2, kt2, T=128)

  idx_full = _topk(score.reshape(H * L, L), num)

  table = v[0].reshape(H * L, D)
  gathered = _gather_rows(table, idx_full, num, H, L)
  return (gathered.reshape(B, H, L, num, D), score.reshape(B, H, L, L))
